# baseline (speedup 1.0000x reference)
import functools

import jax
import jax.numpy as jnp
from jax import lax
from jax.experimental import pallas as pl
from jax.experimental.pallas import tpu as pltpu
from jax.experimental.pallas import tpu_sc as plsc

EMBED = 64
LANES = 16
CHUNK = 128
SEQ = 200
BATCH = 1024
TOTAL = SEQ * BATCH
NCHUNKS = TOTAL // CHUNK
CHUNKS_PER_POS = BATCH // CHUNK


def _build_sc_kernel():
    info = plsc.get_sparse_core_info()
    nc, ns = info.num_cores, info.num_subcores
    nw = nc * ns
    per_w = NCHUNKS // nw

    mesh = plsc.VectorSubcoreMesh(core_axis_name="c", subcore_axis_name="s")

    @functools.partial(
        pl.kernel,
        mesh=mesh,
        compiler_params=pltpu.CompilerParams(use_tc_tiling_on_sc=False),
        out_type=jax.ShapeDtypeStruct((TOTAL, EMBED), jnp.float32),
        scratch_types=[
            pltpu.VMEM((CHUNK,), jnp.int32),
            pltpu.VMEM((CHUNK, EMBED), jnp.float32),
            pltpu.VMEM((EMBED,), jnp.float32),
            pltpu.SemaphoreType.DMA,
        ],
    )
    def sc_kernel(ids_hbm, table_hbm, pos_hbm, out_hbm, idx_v, rows_v, pos_v, sem):
        wid = lax.axis_index("s") * nc + lax.axis_index("c")

        def chunk_body(i, carry):
            c = wid * per_w + i
            s_pos = c >> 3
            pltpu.sync_copy(ids_hbm.at[c], idx_v)
            pltpu.sync_copy(pos_hbm.at[s_pos], pos_v)
            pltpu.async_copy(table_hbm.at[idx_v], rows_v, sem).wait()
            pk = [pos_v[pl.ds(j * LANES, LANES)] * 0.2 for j in range(EMBED // LANES)]

            def row_body(r, rcarry):
                for j in range(EMBED // LANES):
                    v = rows_v[r, pl.ds(j * LANES, LANES)]
                    rows_v[r, pl.ds(j * LANES, LANES)] = v * 0.8 + pk[j]
                return rcarry

            lax.fori_loop(0, CHUNK, row_body, 0)
            pltpu.sync_copy(rows_v, out_hbm.at[pl.ds(c * CHUNK, CHUNK)])
            return carry

        lax.fori_loop(0, per_w, chunk_body, 0)

    return sc_kernel


def kernel(input_ids, table, pos_embedding):
    ids2d = input_ids.reshape(NCHUNKS, CHUNK).astype(jnp.int32)
    out = _build_sc_kernel()(ids2d, table, pos_embedding)
    return out.reshape(SEQ, BATCH, EMBED)
